# Initial kernel scaffold; baseline (speedup 1.0000x reference)
import jax
import jax.numpy as jnp
from jax.experimental import pallas as pl


def kernel(x, es_patterns, es_w1, es_b1, es_w2, es_b2, rt_w1, rt_b1, rt_w2, rt_b2, kp_w1, kp_b1, kp_w2, kp_b2, cf_in_w, cf_in_b, cf_out_w, cf_out_b, cf_fuse_w, cf_fuse_b, de_w1, de_b1, de_w2, de_b2, de_gw, de_gb, hf_w1, hf_b1, hf_w2, hf_b2, lt_w1, lt_b1, lt_w2, lt_b2, sp_in_w, sp_in_b, sp_out_w, sp_out_b, sp_q1_w, sp_q1_b, sp_q2_w, sp_q2_b, tc_w, tc_b, tf_w, tf_b):
    raise NotImplementedError("write your pallas kernel here")


# speedup vs baseline: 3.2124x; 3.2124x over previous
import functools

import jax
import jax.numpy as jnp
from jax.experimental import pallas as pl
from jax.experimental.pallas import tpu as pltpu

D = 1664
NF = 26
FD = 64
NE = 6
H = 416
B_TOK = 1024
TOPC = int(0.2 * D)
KMAX = 4

_F32 = jnp.float32
_BF16 = jnp.bfloat16


def _dot(a, b):
    return jax.lax.dot_general(a, b, (((1,), (0,)), ((), ())),
                               preferred_element_type=_F32)


def _router_kernel(x_ref, espT, esw1T, esb1, esw2T, esb2, rtw1xT,
                   rtw1sT, rtb1, rtw2T, rtb2,
                   kpw1T, kpb1, kpw2T, kpb2, out_ref):
    x = x_ref[...]

    sims = _dot(x, espT[...])
    h1 = jnp.maximum(_dot(x, esw1T[...]) + esb1[...], 0.0)
    spec = _dot(h1, esw2T[...]) + esb2[...]
    ss = jax.nn.sigmoid(sims + spec)
    h2 = jnp.maximum(_dot(x, rtw1xT[...])
                     + _dot(ss, rtw1sT[...])
                     + rtb1[...], 0.0)
    logits = _dot(h2, rtw2T[...]) + rtb2[...]
    m = jnp.max(logits, axis=-1, keepdims=True)
    e = jnp.exp(logits - m)
    probs = e / jnp.sum(e, axis=-1, keepdims=True)

    zc = jnp.sum((x == 0.0).astype(_F32), axis=1, keepdims=True) / D
    mean = jnp.sum(x, axis=1, keepdims=True) / D
    d = x - mean
    var = jnp.sum(d * d, axis=1, keepdims=True) / (D - 1)
    a = jnp.abs(x)
    mag = jnp.max(a, axis=1, keepdims=True)
    nrm = jnp.sqrt(jnp.sum(x * x, axis=1, keepdims=True))
    std = jnp.sqrt(var + 1e-8)
    y = d / std
    skew = jnp.sum(y * y * y, axis=1, keepdims=True) / D

    denom = jnp.sum(a, axis=1, keepdims=True) + 1e-8

    def bis(i, lh):
        lo, hi = lh
        mid = 0.5 * (lo + hi)
        cnt = jnp.sum((a > mid).astype(_F32), axis=1, keepdims=True)
        pred = cnt >= TOPC
        return jnp.where(pred, mid, lo), jnp.where(pred, hi, mid)

    lo, hi = jax.lax.fori_loop(0, 20, bis, (jnp.zeros_like(mag), mag))
    t_est = 0.5 * (lo + hi)
    gt = (a > t_est).astype(_F32)
    c = jnp.sum(gt, axis=1, keepdims=True)
    num = jnp.sum(a * gt, axis=1, keepdims=True) + (TOPC - c) * t_est
    conc = num / denom

    feats = jnp.concatenate([zc, var, mag, nrm, skew, conc], axis=1)
    fz = jnp.maximum(_dot(feats, kpw1T[...]) + kpb1[...], 0.0)
    kr = jax.nn.sigmoid(_dot(fz, kpw2T[...]) + kpb2[...])
    t = 1.0 + 3.0 * kr[:, 0:1]

    pad = jnp.zeros_like(t)
    out_ref[...] = jnp.concatenate([probs, t, pad], axis=1)


def _ksel_kernel(rt_ref, k_ref):
    t = rt_ref[:, 6:7]

    def bis(i, st):
        lo1, hi1, lo2, hi2 = st
        mid1 = 0.5 * (lo1 + hi1)
        mid2 = 0.5 * (lo2 + hi2)
        c1 = jnp.sum((t <= mid1).astype(_F32))
        c2 = jnp.sum((t <= mid2).astype(_F32))
        p1 = c1 >= B_TOK // 2
        p2 = c2 >= B_TOK // 2 + 1
        return (jnp.where(p1, lo1, mid1), jnp.where(p1, mid1, hi1),
                jnp.where(p2, lo2, mid2), jnp.where(p2, mid2, hi2))

    one = jnp.float32(1.0)
    four = jnp.float32(4.0)
    lo1, hi1, lo2, hi2 = jax.lax.fori_loop(
        0, 40, bis, (one, four, one, four))
    med = 0.5 * (hi1 + hi2)
    k_ref[...] = jnp.zeros((1, 1), _F32) + jnp.clip(jnp.floor(med), 1.0, 4.0)


def _cf_kernel(xf_ref, inwT, inb, outwT, outb, fusewT, fuseb, out_ref):
    xf = xf_ref[...]
    xf16 = xf.astype(_BF16)
    qkv = _dot(xf16, inwT[...]) + inb[...]
    q = qkv[:, 0:FD].astype(_BF16)
    kk = qkv[:, FD:2 * FD].astype(_BF16)
    v = qkv[:, 2 * FD:3 * FD].astype(_BF16)

    G = 8
    R = G * NF
    ii = jax.lax.broadcasted_iota(jnp.int32, (R, R), 0) // NF
    jj = jax.lax.broadcasted_iota(jnp.int32, (R, R), 1) // NF
    mask = jnp.where(ii == jj, 0.0, -1e30).astype(_F32)
    scale = 1.0 / (FD ** 0.5)

    for g in range(out_ref.shape[0] // R):
        qg = q[g * R:(g + 1) * R, :]
        kg = kk[g * R:(g + 1) * R, :]
        vg = v[g * R:(g + 1) * R, :]
        s = jax.lax.dot_general(qg, kg, (((1,), (1,)), ((), ())),
                                preferred_element_type=_F32) * scale + mask
        sm = jnp.max(s, axis=1, keepdims=True)
        p = jnp.exp(s - sm)
        p = p / jnp.sum(p, axis=1, keepdims=True)
        out_ref[g * R:(g + 1) * R, :] = _dot(p.astype(_BF16), vg)

    att = out_ref[...]
    ao = _dot(att.astype(_BF16), outwT[...]) + outb[...]
    fused_in = (ao * xf).astype(_BF16)
    out_ref[...] = _dot(fused_in, fusewT[...]) + fuseb[...]


def _experts_kernel(x_ref, cf_ref, probs_ref, k_ref, sc_ref,
                    dew1T, deb1, dew2T, deb2, degwT, degb,
                    hfw1T, hfb1, hfw2T, hfb2,
                    ltw1T, ltb1, ltw2T, ltb2,
                    spvwT, spvb, spowT, spob, spq1T, spq1b, spq2T, spq2b,
                    out_ref):
    x = x_ref[...]
    x16 = x.astype(_BF16)

    kv = k_ref[0]
    ps = [probs_ref[:, e:e + 1] for e in range(NE)]
    w = []
    mx = ps[0]
    for e in range(1, NE):
        mx = jnp.maximum(mx, ps[e])
    sels = []
    for e in range(NE):
        rank = jnp.zeros_like(ps[e])
        for e2 in range(NE):
            if e2 == e:
                continue
            gtr = (ps[e2] > ps[e]).astype(_F32)
            if e2 < e:
                gtr = gtr + ((ps[e2] == ps[e]).astype(_F32))
            rank = rank + gtr
        sels.append((rank < kv).astype(_F32))
    z = jnp.zeros_like(ps[0])
    ge = []
    for e in range(NE):
        g = sels[e] * jnp.exp(ps[e] - mx)
        ge.append(g)
        z = z + g
    for e in range(NE):
        w.append(ge[e] / z)

    h = jnp.maximum(_dot(x16, dew1T[...]) + deb1[...], 0.0)
    t = _dot(h.astype(_BF16), dew2T[...]) + deb2[...]
    g = jax.nn.sigmoid(_dot(x16, degwT[...]) + degb[...])
    acc = w[0] * cf_ref[...] + w[1] * (t + g * x)

    h = jnp.tanh(_dot(x16, hfw1T[...]) + hfb1[...])
    f = _dot(h.astype(_BF16), hfw2T[...]) + hfb2[...]
    acc = acc + w[2] * (x + (f - x) * x)

    zpre = _dot(x16, ltw1T[...]) + ltb1[...]
    h = jnp.where(zpre > 0.0, zpre, jnp.exp(zpre) - 1.0)
    t = _dot(h.astype(_BF16), ltw2T[...]) + ltb2[...]
    acc = acc + w[3] * (jnp.sign(x) * jnp.sqrt(jnp.abs(t * x) + 1e-8))

    v = _dot(x16, spvwT[...]) + spvb[...]
    xa = _dot(v.astype(_BF16), spowT[...]) + spob[...]
    h = jnp.maximum(_dot((xa * x).astype(_BF16), spq1T[...]) + spq1b[...], 0.0)
    sp = _dot(h.astype(_BF16), spq2T[...]) + spq2b[...]
    acc = acc + w[4] * sp

    bt = x.shape[0]
    zcol = jnp.zeros((bt, 1), dtype=_F32)
    xm = jnp.concatenate([zcol, x[:, :D - 1]], axis=1)
    xp = jnp.concatenate([x[:, 1:], zcol], axis=1)
    wa = jnp.zeros_like(x)
    for o in range(4):
        co = sc_ref[o * 3] * xm + sc_ref[o * 3 + 1] * x + sc_ref[o * 3 + 2] * xp
        ro = jnp.maximum(co + sc_ref[12 + o], 0.0)
        wa = wa + sc_ref[16 + o] * ro
    wgt = jax.nn.sigmoid(wa + sc_ref[20])
    acc = acc + w[5] * (x * wgt)

    out_ref[...] = acc


def _full(shape):
    return pl.BlockSpec(shape, lambda i: (0, 0))


@functools.partial(jax.jit, static_argnums=())
def kernel(x, es_patterns, es_w1, es_b1, es_w2, es_b2, rt_w1, rt_b1, rt_w2,
           rt_b2, kp_w1, kp_b1, kp_w2, kp_b2, cf_in_w, cf_in_b, cf_out_w,
           cf_out_b, cf_fuse_w, cf_fuse_b, de_w1, de_b1, de_w2, de_b2, de_gw,
           de_gb, hf_w1, hf_b1, hf_w2, hf_b2, lt_w1, lt_b1, lt_w2, lt_b2,
           sp_in_w, sp_in_b, sp_out_w, sp_out_b, sp_q1_w, sp_q1_b, sp_q2_w,
           sp_q2_b, tc_w, tc_b, tf_w, tf_b):
    f32 = _F32
    x = x.astype(f32)

    BT_R = 256
    grid_r = (B_TOK // BT_R,)
    rt_out = pl.pallas_call(
        _router_kernel,
        grid=grid_r,
        in_specs=[
            pl.BlockSpec((BT_R, D), lambda i: (i, 0)),
            _full((D, NE)),
            _full((D, D // 2)), _full((1, D // 2)),
            _full((D // 2, NE)), _full((1, NE)),
            _full((D, D // 2)),
            _full((NE, D // 2)), _full((1, D // 2)),
            _full((D // 2, NE)), _full((1, NE)),
            _full((NE, 16)), _full((1, 16)), _full((16, 1)), _full((1, 1)),
        ],
        out_specs=pl.BlockSpec((BT_R, 8), lambda i: (i, 0)),
        out_shape=jax.ShapeDtypeStruct((B_TOK, 8), f32),
    )(x, es_patterns.T.astype(f32), es_w1.T.astype(f32),
      es_b1.reshape(1, -1), es_w2.T.astype(f32), es_b2.reshape(1, -1),
      rt_w1[:, :D].T.astype(f32), rt_w1[:, D:].T.astype(f32),
      rt_b1.reshape(1, -1), rt_w2.T.astype(f32), rt_b2.reshape(1, -1),
      kp_w1.T.astype(f32), kp_b1.reshape(1, -1),
      kp_w2.T.astype(f32), kp_b2.reshape(1, -1))

    kval = pl.pallas_call(
        _ksel_kernel,
        out_shape=jax.ShapeDtypeStruct((1, 1), f32),
    )(rt_out)

    xf = x.reshape(B_TOK * NF, FD)
    BT_C = 128 * NF
    cf_out = pl.pallas_call(
        _cf_kernel,
        grid=(B_TOK * NF // BT_C,),
        in_specs=[
            pl.BlockSpec((BT_C, FD), lambda i: (i, 0)),
            _full((FD, 3 * FD)), _full((1, 3 * FD)),
            _full((FD, FD)), _full((1, FD)),
            _full((FD, FD)), _full((1, FD)),
        ],
        out_specs=pl.BlockSpec((BT_C, FD), lambda i: (i, 0)),
        out_shape=jax.ShapeDtypeStruct((B_TOK * NF, FD), f32),
    )(xf, cf_in_w.T.astype(_BF16), cf_in_b.reshape(1, -1),
      cf_out_w.T.astype(_BF16), cf_out_b.reshape(1, -1),
      cf_fuse_w.T.astype(_BF16), cf_fuse_b.reshape(1, -1))
    cfr = cf_out.reshape(B_TOK, D)

    sc = jnp.concatenate([tc_w.reshape(-1), tc_b.reshape(-1),
                          tf_w.reshape(-1), tf_b.reshape(-1)]).astype(f32)
    BT_E = 128
    bspec = lambda shape: pl.BlockSpec(shape, lambda i: (0, 0))
    out = pl.pallas_call(
        _experts_kernel,
        grid=(B_TOK // BT_E,),
        in_specs=[
            pl.BlockSpec((BT_E, D), lambda i: (i, 0)),
            pl.BlockSpec((BT_E, D), lambda i: (i, 0)),
            pl.BlockSpec((BT_E, 8), lambda i: (i, 0)),
            pl.BlockSpec(memory_space=pltpu.SMEM),
            pl.BlockSpec(memory_space=pltpu.SMEM),
            bspec((D, H)), bspec((1, H)), bspec((H, D)), bspec((1, D)),
            bspec((D, D)), bspec((1, D)),
            bspec((D, H)), bspec((1, H)), bspec((H, D)), bspec((1, D)),
            bspec((D, H)), bspec((1, H)), bspec((H, D)), bspec((1, D)),
            bspec((D, D)), bspec((1, D)), bspec((D, D)), bspec((1, D)),
            bspec((D, D)), bspec((1, D)), bspec((D, D)), bspec((1, D)),
        ],
        out_specs=pl.BlockSpec((BT_E, D), lambda i: (i, 0)),
        out_shape=jax.ShapeDtypeStruct((B_TOK, D), f32),
    )(x, cfr, rt_out, kval.reshape(-1), sc,
      de_w1.T.astype(_BF16), de_b1.reshape(1, -1),
      de_w2.T.astype(_BF16), de_b2.reshape(1, -1),
      de_gw.T.astype(_BF16), de_gb.reshape(1, -1),
      hf_w1.T.astype(_BF16), hf_b1.reshape(1, -1),
      hf_w2.T.astype(_BF16), hf_b2.reshape(1, -1),
      lt_w1.T.astype(_BF16), lt_b1.reshape(1, -1),
      lt_w2.T.astype(_BF16), lt_b2.reshape(1, -1),
      sp_in_w[2 * D:, :].T.astype(_BF16), sp_in_b[2 * D:].reshape(1, -1),
      sp_out_w.T.astype(_BF16), sp_out_b.reshape(1, -1),
      sp_q1_w.T.astype(_BF16), sp_q1_b.reshape(1, -1),
      sp_q2_w.T.astype(_BF16), sp_q2_b.reshape(1, -1))
    return out


# speedup vs baseline: 3.4364x; 1.0697x over previous
import functools

import jax
import jax.numpy as jnp
from jax.experimental import pallas as pl
from jax.experimental.pallas import tpu as pltpu

D = 1664
NF = 26
FD = 64
NE = 6
H = 416
B_TOK = 1024
TOPC = int(0.2 * D)
KMAX = 4

_F32 = jnp.float32
_BF16 = jnp.bfloat16


def _dot(a, b):
    return jax.lax.dot_general(a, b, (((1,), (0,)), ((), ())),
                               preferred_element_type=_F32)


def _router_kernel(x_ref, espT, esw1T, esb1, esw2T, esb2, rtw1xT,
                   rtw1sT, rtb1, rtw2T, rtb2,
                   kpw1T, kpb1, kpw2T, kpb2, out_ref):
    x = x_ref[...]

    sims = _dot(x, espT[...])
    h1 = jnp.maximum(_dot(x, esw1T[...]) + esb1[...], 0.0)
    spec = _dot(h1, esw2T[...]) + esb2[...]
    ss = jax.nn.sigmoid(sims + spec)
    h2 = jnp.maximum(_dot(x, rtw1xT[...])
                     + _dot(ss, rtw1sT[...])
                     + rtb1[...], 0.0)
    logits = _dot(h2, rtw2T[...]) + rtb2[...]
    m = jnp.max(logits, axis=-1, keepdims=True)
    e = jnp.exp(logits - m)
    probs = e / jnp.sum(e, axis=-1, keepdims=True)

    zc = jnp.sum((x == 0.0).astype(_F32), axis=1, keepdims=True) / D
    mean = jnp.sum(x, axis=1, keepdims=True) / D
    d = x - mean
    var = jnp.sum(d * d, axis=1, keepdims=True) / (D - 1)
    a = jnp.abs(x)
    mag = jnp.max(a, axis=1, keepdims=True)
    nrm = jnp.sqrt(jnp.sum(x * x, axis=1, keepdims=True))
    std = jnp.sqrt(var + 1e-8)
    y = d / std
    skew = jnp.sum(y * y * y, axis=1, keepdims=True) / D

    denom = jnp.sum(a, axis=1, keepdims=True) + 1e-8

    def bis(i, lh):
        lo, hi = lh
        mid = 0.5 * (lo + hi)
        cnt = jnp.sum((a > mid).astype(_F32), axis=1, keepdims=True)
        pred = cnt >= TOPC
        return jnp.where(pred, mid, lo), jnp.where(pred, hi, mid)

    lo, hi = jax.lax.fori_loop(0, 20, bis, (jnp.zeros_like(mag), mag))
    t_est = 0.5 * (lo + hi)
    gt = (a > t_est).astype(_F32)
    c = jnp.sum(gt, axis=1, keepdims=True)
    num = jnp.sum(a * gt, axis=1, keepdims=True) + (TOPC - c) * t_est
    conc = num / denom

    feats = jnp.concatenate([zc, var, mag, nrm, skew, conc], axis=1)
    fz = jnp.maximum(_dot(feats, kpw1T[...]) + kpb1[...], 0.0)
    kr = jax.nn.sigmoid(_dot(fz, kpw2T[...]) + kpb2[...])
    t = 1.0 + 3.0 * kr[:, 0:1]

    pad = jnp.zeros_like(t)
    out_ref[...] = jnp.concatenate([probs, t, pad], axis=1)


def _ksel_kernel(rt_ref, k_ref):
    t = rt_ref[:, 6:7]
    T = 2048
    iot = jax.lax.broadcasted_iota(jnp.int32, (1, T), 1).astype(_F32) + 0.5

    def order_stat(target):
        thr = 1.0 + 3.0 * iot / T
        counts = jnp.sum((t <= thr).astype(_F32), axis=0, keepdims=True)
        idx = jnp.sum((counts < target).astype(_F32))
        blo = 1.0 + 3.0 * (idx - 0.5) / T
        thr2 = blo + (3.0 / T) * iot / T
        counts2 = jnp.sum((t <= thr2).astype(_F32), axis=0, keepdims=True)
        idx2 = jnp.sum((counts2 < target).astype(_F32))
        return blo + (3.0 / T) * (idx2 + 0.5) / T

    med = 0.5 * (order_stat(B_TOK // 2) + order_stat(B_TOK // 2 + 1))
    k_ref[...] = jnp.zeros((1, 1), _F32) + jnp.clip(jnp.floor(med), 1.0, 4.0)


def _cf_kernel(xf_ref, inwT, inb, outwT, outb, fusewT, fuseb, out_ref):
    xf = xf_ref[...]
    xf16 = xf.astype(_BF16)
    qkv = _dot(xf16, inwT[...]) + inb[...]
    q = qkv[:, 0:FD].astype(_BF16)
    kk = qkv[:, FD:2 * FD].astype(_BF16)
    v = qkv[:, 2 * FD:3 * FD].astype(_BF16)

    G = 8
    R = G * NF
    ii = jax.lax.broadcasted_iota(jnp.int32, (R, R), 0) // NF
    jj = jax.lax.broadcasted_iota(jnp.int32, (R, R), 1) // NF
    mask = jnp.where(ii == jj, 0.0, -1e30).astype(_F32)
    scale = 1.0 / (FD ** 0.5)

    atts = []
    for g in range(xf.shape[0] // R):
        qg = q[g * R:(g + 1) * R, :]
        kg = kk[g * R:(g + 1) * R, :]
        vg = v[g * R:(g + 1) * R, :]
        s = jax.lax.dot_general(qg, kg, (((1,), (1,)), ((), ())),
                                preferred_element_type=_F32) * scale + mask
        sm = jnp.max(s, axis=1, keepdims=True)
        p = jnp.exp(s - sm)
        p = p / jnp.sum(p, axis=1, keepdims=True)
        atts.append(_dot(p.astype(_BF16), vg))

    att = jnp.concatenate(atts, axis=0)
    ao = _dot(att.astype(_BF16), outwT[...]) + outb[...]
    fused_in = (ao * xf).astype(_BF16)
    out_ref[...] = _dot(fused_in, fusewT[...]) + fuseb[...]


def _experts_kernel(x_ref, cf_ref, probs_ref, k_ref, sc_ref,
                    dew1T, deb1, dew2T, deb2, degwT, degb,
                    hfw1T, hfb1, hfw2T, hfb2,
                    ltw1T, ltb1, ltw2T, ltb2,
                    spvwT, spvb, spowT, spob, spq1T, spq1b, spq2T, spq2b,
                    out_ref):
    x = x_ref[...]
    x16 = x.astype(_BF16)

    kv = k_ref[0]
    ps = [probs_ref[:, e:e + 1] for e in range(NE)]
    w = []
    mx = ps[0]
    for e in range(1, NE):
        mx = jnp.maximum(mx, ps[e])
    sels = []
    for e in range(NE):
        rank = jnp.zeros_like(ps[e])
        for e2 in range(NE):
            if e2 == e:
                continue
            gtr = (ps[e2] > ps[e]).astype(_F32)
            if e2 < e:
                gtr = gtr + ((ps[e2] == ps[e]).astype(_F32))
            rank = rank + gtr
        sels.append((rank < kv).astype(_F32))
    z = jnp.zeros_like(ps[0])
    ge = []
    for e in range(NE):
        g = sels[e] * jnp.exp(ps[e] - mx)
        ge.append(g)
        z = z + g
    for e in range(NE):
        w.append(ge[e] / z)

    h = jnp.maximum(_dot(x16, dew1T[...]) + deb1[...], 0.0)
    t = _dot(h.astype(_BF16), dew2T[...]) + deb2[...]
    g = jax.nn.sigmoid(_dot(x16, degwT[...]) + degb[...])
    acc = w[0] * cf_ref[...] + w[1] * (t + g * x)

    h = jnp.tanh(_dot(x16, hfw1T[...]) + hfb1[...])
    f = _dot(h.astype(_BF16), hfw2T[...]) + hfb2[...]
    acc = acc + w[2] * (x + (f - x) * x)

    zpre = _dot(x16, ltw1T[...]) + ltb1[...]
    h = jnp.where(zpre > 0.0, zpre, jnp.exp(zpre) - 1.0)
    t = _dot(h.astype(_BF16), ltw2T[...]) + ltb2[...]
    acc = acc + w[3] * (jnp.sign(x) * jnp.sqrt(jnp.abs(t * x) + 1e-8))

    v = _dot(x16, spvwT[...]) + spvb[...]
    xa = _dot(v.astype(_BF16), spowT[...]) + spob[...]
    h = jnp.maximum(_dot((xa * x).astype(_BF16), spq1T[...]) + spq1b[...], 0.0)
    sp = _dot(h.astype(_BF16), spq2T[...]) + spq2b[...]
    acc = acc + w[4] * sp

    bt = x.shape[0]
    zcol = jnp.zeros((bt, 1), dtype=_F32)
    xm = jnp.concatenate([zcol, x[:, :D - 1]], axis=1)
    xp = jnp.concatenate([x[:, 1:], zcol], axis=1)
    wa = jnp.zeros_like(x)
    for o in range(4):
        co = sc_ref[o * 3] * xm + sc_ref[o * 3 + 1] * x + sc_ref[o * 3 + 2] * xp
        ro = jnp.maximum(co + sc_ref[12 + o], 0.0)
        wa = wa + sc_ref[16 + o] * ro
    wgt = jax.nn.sigmoid(wa + sc_ref[20])
    acc = acc + w[5] * (x * wgt)

    out_ref[...] = acc


def _full(shape):
    return pl.BlockSpec(shape, lambda i: (0, 0))


@functools.partial(jax.jit, static_argnums=())
def kernel(x, es_patterns, es_w1, es_b1, es_w2, es_b2, rt_w1, rt_b1, rt_w2,
           rt_b2, kp_w1, kp_b1, kp_w2, kp_b2, cf_in_w, cf_in_b, cf_out_w,
           cf_out_b, cf_fuse_w, cf_fuse_b, de_w1, de_b1, de_w2, de_b2, de_gw,
           de_gb, hf_w1, hf_b1, hf_w2, hf_b2, lt_w1, lt_b1, lt_w2, lt_b2,
           sp_in_w, sp_in_b, sp_out_w, sp_out_b, sp_q1_w, sp_q1_b, sp_q2_w,
           sp_q2_b, tc_w, tc_b, tf_w, tf_b):
    f32 = _F32
    x = x.astype(f32)

    BT_R = 256
    grid_r = (B_TOK // BT_R,)
    rt_out = pl.pallas_call(
        _router_kernel,
        grid=grid_r,
        in_specs=[
            pl.BlockSpec((BT_R, D), lambda i: (i, 0)),
            _full((D, NE)),
            _full((D, D // 2)), _full((1, D // 2)),
            _full((D // 2, NE)), _full((1, NE)),
            _full((D, D // 2)),
            _full((NE, D // 2)), _full((1, D // 2)),
            _full((D // 2, NE)), _full((1, NE)),
            _full((NE, 16)), _full((1, 16)), _full((16, 1)), _full((1, 1)),
        ],
        out_specs=pl.BlockSpec((BT_R, 8), lambda i: (i, 0)),
        out_shape=jax.ShapeDtypeStruct((B_TOK, 8), f32),
    )(x, es_patterns.T.astype(f32), es_w1.T.astype(f32),
      es_b1.reshape(1, -1), es_w2.T.astype(f32), es_b2.reshape(1, -1),
      rt_w1[:, :D].T.astype(f32), rt_w1[:, D:].T.astype(f32),
      rt_b1.reshape(1, -1), rt_w2.T.astype(f32), rt_b2.reshape(1, -1),
      kp_w1.T.astype(f32), kp_b1.reshape(1, -1),
      kp_w2.T.astype(f32), kp_b2.reshape(1, -1))

    kval = pl.pallas_call(
        _ksel_kernel,
        out_shape=jax.ShapeDtypeStruct((1, 1), f32),
    )(rt_out)

    xf = x.reshape(B_TOK * NF, FD)
    BT_C = 128 * NF
    cf_out = pl.pallas_call(
        _cf_kernel,
        grid=(B_TOK * NF // BT_C,),
        in_specs=[
            pl.BlockSpec((BT_C, FD), lambda i: (i, 0)),
            _full((FD, 3 * FD)), _full((1, 3 * FD)),
            _full((FD, FD)), _full((1, FD)),
            _full((FD, FD)), _full((1, FD)),
        ],
        out_specs=pl.BlockSpec((BT_C, FD), lambda i: (i, 0)),
        out_shape=jax.ShapeDtypeStruct((B_TOK * NF, FD), f32),
    )(xf, cf_in_w.T.astype(_BF16), cf_in_b.reshape(1, -1),
      cf_out_w.T.astype(_BF16), cf_out_b.reshape(1, -1),
      cf_fuse_w.T.astype(_BF16), cf_fuse_b.reshape(1, -1))
    cfr = cf_out.reshape(B_TOK, D)

    sc = jnp.concatenate([tc_w.reshape(-1), tc_b.reshape(-1),
                          tf_w.reshape(-1), tf_b.reshape(-1)]).astype(f32)
    BT_E = 256
    bspec = lambda shape: pl.BlockSpec(shape, lambda i: (0, 0))
    out = pl.pallas_call(
        _experts_kernel,
        grid=(B_TOK // BT_E,),
        in_specs=[
            pl.BlockSpec((BT_E, D), lambda i: (i, 0)),
            pl.BlockSpec((BT_E, D), lambda i: (i, 0)),
            pl.BlockSpec((BT_E, 8), lambda i: (i, 0)),
            pl.BlockSpec(memory_space=pltpu.SMEM),
            pl.BlockSpec(memory_space=pltpu.SMEM),
            bspec((D, H)), bspec((1, H)), bspec((H, D)), bspec((1, D)),
            bspec((D, D)), bspec((1, D)),
            bspec((D, H)), bspec((1, H)), bspec((H, D)), bspec((1, D)),
            bspec((D, H)), bspec((1, H)), bspec((H, D)), bspec((1, D)),
            bspec((D, D)), bspec((1, D)), bspec((D, D)), bspec((1, D)),
            bspec((D, D)), bspec((1, D)), bspec((D, D)), bspec((1, D)),
        ],
        out_specs=pl.BlockSpec((BT_E, D), lambda i: (i, 0)),
        out_shape=jax.ShapeDtypeStruct((B_TOK, D), f32),
    )(x, cfr, rt_out, kval.reshape(-1), sc,
      de_w1.T.astype(_BF16), de_b1.reshape(1, -1),
      de_w2.T.astype(_BF16), de_b2.reshape(1, -1),
      de_gw.T.astype(_BF16), de_gb.reshape(1, -1),
      hf_w1.T.astype(_BF16), hf_b1.reshape(1, -1),
      hf_w2.T.astype(_BF16), hf_b2.reshape(1, -1),
      lt_w1.T.astype(_BF16), lt_b1.reshape(1, -1),
      lt_w2.T.astype(_BF16), lt_b2.reshape(1, -1),
      sp_in_w[2 * D:, :].T.astype(_BF16), sp_in_b[2 * D:].reshape(1, -1),
      sp_out_w.T.astype(_BF16), sp_out_b.reshape(1, -1),
      sp_q1_w.T.astype(_BF16), sp_q1_b.reshape(1, -1),
      sp_q2_w.T.astype(_BF16), sp_q2_b.reshape(1, -1))
    return out


# speedup vs baseline: 3.9785x; 1.1578x over previous
import functools

import jax
import jax.numpy as jnp
from jax.experimental import pallas as pl
from jax.experimental.pallas import tpu as pltpu

D = 1664
NF = 26
FD = 64
NE = 6
H = 416
B_TOK = 1024
TOPC = int(0.2 * D)
KMAX = 4

_F32 = jnp.float32
_BF16 = jnp.bfloat16


def _dot(a, b):
    return jax.lax.dot_general(a, b, (((1,), (0,)), ((), ())),
                               preferred_element_type=_F32)


def _dotT(a, w):
    return jax.lax.dot_general(a, w, (((1,), (1,)), ((), ())),
                               preferred_element_type=_F32)


def _router_kernel(x_ref, esp, esw1, esb1, esw2, esb2, rtw1x,
                   rtw1s, rtb1, rtw2, rtb2,
                   kpw1, kpb1, kpw2T, kpb2, out_ref):
    x = x_ref[...]

    sims = _dotT(x, esp[...])
    h1 = jnp.maximum(_dotT(x, esw1[...]) + esb1[...], 0.0)
    spec = _dotT(h1, esw2[...]) + esb2[...]
    ss = jax.nn.sigmoid(sims + spec)
    h2 = jnp.maximum(_dotT(x, rtw1x[...])
                     + _dotT(ss, rtw1s[...])
                     + rtb1[...], 0.0)
    logits = _dotT(h2, rtw2[...]) + rtb2[...]
    m = jnp.max(logits, axis=-1, keepdims=True)
    e = jnp.exp(logits - m)
    probs = e / jnp.sum(e, axis=-1, keepdims=True)

    zc = jnp.sum((x == 0.0).astype(_F32), axis=1, keepdims=True) / D
    mean = jnp.sum(x, axis=1, keepdims=True) / D
    d = x - mean
    var = jnp.sum(d * d, axis=1, keepdims=True) / (D - 1)
    a = jnp.abs(x)
    mag = jnp.max(a, axis=1, keepdims=True)
    nrm = jnp.sqrt(jnp.sum(x * x, axis=1, keepdims=True))
    std = jnp.sqrt(var + 1e-8)
    y = d / std
    skew = jnp.sum(y * y * y, axis=1, keepdims=True) / D

    denom = jnp.sum(a, axis=1, keepdims=True) + 1e-8

    def bis(i, lh):
        lo, hi = lh
        mid = 0.5 * (lo + hi)
        cnt = jnp.sum((a > mid).astype(_F32), axis=1, keepdims=True)
        pred = cnt >= TOPC
        return jnp.where(pred, mid, lo), jnp.where(pred, hi, mid)

    lo, hi = jax.lax.fori_loop(0, 20, bis, (jnp.zeros_like(mag), mag))
    t_est = 0.5 * (lo + hi)
    gt = (a > t_est).astype(_F32)
    c = jnp.sum(gt, axis=1, keepdims=True)
    num = jnp.sum(a * gt, axis=1, keepdims=True) + (TOPC - c) * t_est
    conc = num / denom

    feats = jnp.concatenate([zc, var, mag, nrm, skew, conc], axis=1)
    fz = jnp.maximum(_dotT(feats, kpw1[...]) + kpb1[...], 0.0)
    kr = jax.nn.sigmoid(_dot(fz, kpw2T[...]) + kpb2[...])
    t = 1.0 + 3.0 * kr[:, 0:1]

    pad = jnp.zeros_like(t)
    out_ref[...] = jnp.concatenate([probs, t, pad], axis=1)


def _ksel_kernel(rt_ref, k_ref):
    t = rt_ref[:, 6:7]
    T = 2048
    iot = jax.lax.broadcasted_iota(jnp.int32, (1, T), 1).astype(_F32) + 0.5

    def order_stat(target):
        thr = 1.0 + 3.0 * iot / T
        counts = jnp.sum((t <= thr).astype(_F32), axis=0, keepdims=True)
        idx = jnp.sum((counts < target).astype(_F32))
        blo = 1.0 + 3.0 * (idx - 0.5) / T
        thr2 = blo + (3.0 / T) * iot / T
        counts2 = jnp.sum((t <= thr2).astype(_F32), axis=0, keepdims=True)
        idx2 = jnp.sum((counts2 < target).astype(_F32))
        return blo + (3.0 / T) * (idx2 + 0.5) / T

    med = 0.5 * (order_stat(B_TOK // 2) + order_stat(B_TOK // 2 + 1))
    k_ref[...] = jnp.zeros((1, 1), _F32) + jnp.clip(jnp.floor(med), 1.0, 4.0)


def _cf_kernel(xf_ref, inwT, inb, outwT, outb, fusewT, fuseb, out_ref):
    xf = xf_ref[...]
    xf16 = xf.astype(_BF16)
    qkv = _dot(xf16, inwT[...]) + inb[...]
    q = qkv[:, 0:FD].astype(_BF16)
    kk = qkv[:, FD:2 * FD].astype(_BF16)
    v = qkv[:, 2 * FD:3 * FD].astype(_BF16)

    G = 8
    R = G * NF
    ii = jax.lax.broadcasted_iota(jnp.int32, (R, R), 0) // NF
    jj = jax.lax.broadcasted_iota(jnp.int32, (R, R), 1) // NF
    mask = jnp.where(ii == jj, 0.0, -1e30).astype(_F32)
    scale = 1.0 / (FD ** 0.5)

    atts = []
    for g in range(xf.shape[0] // R):
        qg = q[g * R:(g + 1) * R, :]
        kg = kk[g * R:(g + 1) * R, :]
        vg = v[g * R:(g + 1) * R, :]
        s = jax.lax.dot_general(qg, kg, (((1,), (1,)), ((), ())),
                                preferred_element_type=_F32) * scale + mask
        sm = jnp.max(s, axis=1, keepdims=True)
        p = jnp.exp(s - sm)
        p = p / jnp.sum(p, axis=1, keepdims=True)
        atts.append(_dot(p.astype(_BF16), vg))

    att = jnp.concatenate(atts, axis=0)
    ao = _dot(att.astype(_BF16), outwT[...]) + outb[...]
    fused_in = (ao * xf).astype(_BF16)
    out_ref[...] = _dot(fused_in, fusewT[...]) + fuseb[...]


def _experts_kernel(x_ref, cf_ref, probs_ref, k_ref, sc_ref,
                    dew1, deb1, dew2, deb2, degw, degb,
                    hfw1, hfb1, hfw2, hfb2,
                    ltw1, ltb1, ltw2, ltb2,
                    spvw, spvb, spow, spob, spq1, spq1b, spq2, spq2b,
                    out_ref):
    x = x_ref[...]
    x16 = x.astype(_BF16)

    kv = k_ref[0]
    ps = [probs_ref[:, e:e + 1] for e in range(NE)]
    w = []
    mx = ps[0]
    for e in range(1, NE):
        mx = jnp.maximum(mx, ps[e])
    sels = []
    for e in range(NE):
        rank = jnp.zeros_like(ps[e])
        for e2 in range(NE):
            if e2 == e:
                continue
            gtr = (ps[e2] > ps[e]).astype(_F32)
            if e2 < e:
                gtr = gtr + ((ps[e2] == ps[e]).astype(_F32))
            rank = rank + gtr
        sels.append((rank < kv).astype(_F32))
    z = jnp.zeros_like(ps[0])
    ge = []
    for e in range(NE):
        g = sels[e] * jnp.exp(ps[e] - mx)
        ge.append(g)
        z = z + g
    for e in range(NE):
        w.append(ge[e] / z)

    h = jnp.maximum(_dotT(x16, dew1[...]) + deb1[...], 0.0)
    t = _dotT(h.astype(_BF16), dew2[...]) + deb2[...]
    g = jax.nn.sigmoid(_dotT(x16, degw[...]) + degb[...])
    acc = w[0] * cf_ref[...] + w[1] * (t + g * x)

    h = jnp.tanh(_dotT(x16, hfw1[...]) + hfb1[...])
    f = _dotT(h.astype(_BF16), hfw2[...]) + hfb2[...]
    acc = acc + w[2] * (x + (f - x) * x)

    zpre = _dotT(x16, ltw1[...]) + ltb1[...]
    h = jnp.where(zpre > 0.0, zpre, jnp.exp(zpre) - 1.0)
    t = _dotT(h.astype(_BF16), ltw2[...]) + ltb2[...]
    acc = acc + w[3] * (jnp.sign(x) * jnp.sqrt(jnp.abs(t * x) + 1e-8))

    v = _dotT(x16, spvw[...]) + spvb[...]
    xa = _dotT(v.astype(_BF16), spow[...]) + spob[...]
    h = jnp.maximum(_dotT((xa * x).astype(_BF16), spq1[...]) + spq1b[...], 0.0)
    sp = _dotT(h.astype(_BF16), spq2[...]) + spq2b[...]
    acc = acc + w[4] * sp

    bt = x.shape[0]
    zcol = jnp.zeros((bt, 1), dtype=_F32)
    xm = jnp.concatenate([zcol, x[:, :D - 1]], axis=1)
    xp = jnp.concatenate([x[:, 1:], zcol], axis=1)
    wa = jnp.zeros_like(x)
    for o in range(4):
        co = sc_ref[o * 3] * xm + sc_ref[o * 3 + 1] * x + sc_ref[o * 3 + 2] * xp
        ro = jnp.maximum(co + sc_ref[12 + o], 0.0)
        wa = wa + sc_ref[16 + o] * ro
    wgt = jax.nn.sigmoid(wa + sc_ref[20])
    acc = acc + w[5] * (x * wgt)

    out_ref[...] = acc


def _full(shape):
    return pl.BlockSpec(shape, lambda i: (0, 0))


@functools.partial(jax.jit, static_argnums=())
def kernel(x, es_patterns, es_w1, es_b1, es_w2, es_b2, rt_w1, rt_b1, rt_w2,
           rt_b2, kp_w1, kp_b1, kp_w2, kp_b2, cf_in_w, cf_in_b, cf_out_w,
           cf_out_b, cf_fuse_w, cf_fuse_b, de_w1, de_b1, de_w2, de_b2, de_gw,
           de_gb, hf_w1, hf_b1, hf_w2, hf_b2, lt_w1, lt_b1, lt_w2, lt_b2,
           sp_in_w, sp_in_b, sp_out_w, sp_out_b, sp_q1_w, sp_q1_b, sp_q2_w,
           sp_q2_b, tc_w, tc_b, tf_w, tf_b):
    f32 = _F32
    x = x.astype(f32)

    BT_R = 256
    grid_r = (B_TOK // BT_R,)
    rt_out = pl.pallas_call(
        _router_kernel,
        grid=grid_r,
        in_specs=[
            pl.BlockSpec((BT_R, D), lambda i: (i, 0)),
            _full((NE, D)),
            _full((D // 2, D)), _full((1, D // 2)),
            _full((NE, D // 2)), _full((1, NE)),
            _full((D // 2, D)),
            _full((D // 2, NE)), _full((1, D // 2)),
            _full((NE, D // 2)), _full((1, NE)),
            _full((16, NE)), _full((1, 16)), _full((16, 1)), _full((1, 1)),
        ],
        out_specs=pl.BlockSpec((BT_R, 8), lambda i: (i, 0)),
        out_shape=jax.ShapeDtypeStruct((B_TOK, 8), f32),
    )(x, es_patterns, es_w1,
      es_b1.reshape(1, -1), es_w2, es_b2.reshape(1, -1),
      rt_w1[:, :D], rt_w1[:, D:],
      rt_b1.reshape(1, -1), rt_w2, rt_b2.reshape(1, -1),
      kp_w1, kp_b1.reshape(1, -1),
      kp_w2.T, kp_b2.reshape(1, -1))

    kval = pl.pallas_call(
        _ksel_kernel,
        out_shape=jax.ShapeDtypeStruct((1, 1), f32),
    )(rt_out)

    xf = x.reshape(B_TOK * NF, FD)
    BT_C = 128 * NF
    cf_out = pl.pallas_call(
        _cf_kernel,
        grid=(B_TOK * NF // BT_C,),
        in_specs=[
            pl.BlockSpec((BT_C, FD), lambda i: (i, 0)),
            _full((FD, 3 * FD)), _full((1, 3 * FD)),
            _full((FD, FD)), _full((1, FD)),
            _full((FD, FD)), _full((1, FD)),
        ],
        out_specs=pl.BlockSpec((BT_C, FD), lambda i: (i, 0)),
        out_shape=jax.ShapeDtypeStruct((B_TOK * NF, FD), f32),
    )(xf, cf_in_w.T.astype(_BF16), cf_in_b.reshape(1, -1),
      cf_out_w.T.astype(_BF16), cf_out_b.reshape(1, -1),
      cf_fuse_w.T.astype(_BF16), cf_fuse_b.reshape(1, -1))
    cfr = cf_out.reshape(B_TOK, D)

    sc = jnp.concatenate([tc_w.reshape(-1), tc_b.reshape(-1),
                          tf_w.reshape(-1), tf_b.reshape(-1)]).astype(f32)
    BT_E = 256
    bspec = lambda shape: pl.BlockSpec(shape, lambda i: (0, 0))
    out = pl.pallas_call(
        _experts_kernel,
        grid=(B_TOK // BT_E,),
        in_specs=[
            pl.BlockSpec((BT_E, D), lambda i: (i, 0)),
            pl.BlockSpec((BT_E, D), lambda i: (i, 0)),
            pl.BlockSpec((BT_E, 8), lambda i: (i, 0)),
            pl.BlockSpec(memory_space=pltpu.SMEM),
            pl.BlockSpec(memory_space=pltpu.SMEM),
            bspec((H, D)), bspec((1, H)), bspec((D, H)), bspec((1, D)),
            bspec((D, D)), bspec((1, D)),
            bspec((H, D)), bspec((1, H)), bspec((D, H)), bspec((1, D)),
            bspec((H, D)), bspec((1, H)), bspec((D, H)), bspec((1, D)),
            bspec((D, D)), bspec((1, D)), bspec((D, D)), bspec((1, D)),
            bspec((D, D)), bspec((1, D)), bspec((D, D)), bspec((1, D)),
        ],
        out_specs=pl.BlockSpec((BT_E, D), lambda i: (i, 0)),
        out_shape=jax.ShapeDtypeStruct((B_TOK, D), f32),
    )(x, cfr, rt_out, kval.reshape(-1), sc,
      de_w1.astype(_BF16), de_b1.reshape(1, -1),
      de_w2.astype(_BF16), de_b2.reshape(1, -1),
      de_gw.astype(_BF16), de_gb.reshape(1, -1),
      hf_w1.astype(_BF16), hf_b1.reshape(1, -1),
      hf_w2.astype(_BF16), hf_b2.reshape(1, -1),
      lt_w1.astype(_BF16), lt_b1.reshape(1, -1),
      lt_w2.astype(_BF16), lt_b2.reshape(1, -1),
      sp_in_w[2 * D:, :].astype(_BF16), sp_in_b[2 * D:].reshape(1, -1),
      sp_out_w.astype(_BF16), sp_out_b.reshape(1, -1),
      sp_q1_w.astype(_BF16), sp_q1_b.reshape(1, -1),
      sp_q2_w.astype(_BF16), sp_q2_b.reshape(1, -1))
    return out


# speedup vs baseline: 3.9994x; 1.0052x over previous
import functools

import jax
import jax.numpy as jnp
from jax.experimental import pallas as pl
from jax.experimental.pallas import tpu as pltpu

D = 1664
NF = 26
FD = 64
NE = 6
H = 416
B_TOK = 1024
TOPC = int(0.2 * D)
KMAX = 4

_F32 = jnp.float32
_BF16 = jnp.bfloat16


def _dot(a, b):
    return jax.lax.dot_general(a, b, (((1,), (0,)), ((), ())),
                               preferred_element_type=_F32)


def _dotT(a, w):
    return jax.lax.dot_general(a, w, (((1,), (1,)), ((), ())),
                               preferred_element_type=_F32)


def _router_kernel(x_ref, esp, esw1, esb1, esw2, esb2, rtw1x,
                   rtw1s, rtb1, rtw2, rtb2,
                   kpw1, kpb1, kpw2T, kpb2, out_ref):
    x = x_ref[...]

    sims = _dotT(x, esp[...])
    h1 = jnp.maximum(_dotT(x, esw1[...]) + esb1[...], 0.0)
    spec = _dotT(h1, esw2[...]) + esb2[...]
    ss = jax.nn.sigmoid(sims + spec)
    h2 = jnp.maximum(_dotT(x, rtw1x[...])
                     + _dotT(ss, rtw1s[...])
                     + rtb1[...], 0.0)
    logits = _dotT(h2, rtw2[...]) + rtb2[...]
    m = jnp.max(logits, axis=-1, keepdims=True)
    e = jnp.exp(logits - m)
    probs = e / jnp.sum(e, axis=-1, keepdims=True)

    zc = jnp.sum((x == 0.0).astype(_F32), axis=1, keepdims=True) / D
    mean = jnp.sum(x, axis=1, keepdims=True) / D
    d = x - mean
    var = jnp.sum(d * d, axis=1, keepdims=True) / (D - 1)
    a = jnp.abs(x)
    mag = jnp.max(a, axis=1, keepdims=True)
    nrm = jnp.sqrt(jnp.sum(x * x, axis=1, keepdims=True))
    std = jnp.sqrt(var + 1e-8)
    y = d / std
    skew = jnp.sum(y * y * y, axis=1, keepdims=True) / D

    denom = jnp.sum(a, axis=1, keepdims=True) + 1e-8

    def bis(i, lh):
        lo, hi = lh
        mid = 0.5 * (lo + hi)
        cnt = jnp.sum((a > mid).astype(_F32), axis=1, keepdims=True)
        pred = cnt >= TOPC
        return jnp.where(pred, mid, lo), jnp.where(pred, hi, mid)

    lo, hi = jax.lax.fori_loop(0, 12, bis, (jnp.zeros_like(mag), mag))
    t_est = 0.5 * (lo + hi)
    gt = (a > t_est).astype(_F32)
    c = jnp.sum(gt, axis=1, keepdims=True)
    num = jnp.sum(a * gt, axis=1, keepdims=True) + (TOPC - c) * t_est
    conc = num / denom

    feats = jnp.concatenate([zc, var, mag, nrm, skew, conc], axis=1)
    fz = jnp.maximum(_dotT(feats, kpw1[...]) + kpb1[...], 0.0)
    kr = jax.nn.sigmoid(_dot(fz, kpw2T[...]) + kpb2[...])
    t = 1.0 + 3.0 * kr[:, 0:1]

    pad = jnp.zeros_like(t)
    out_ref[...] = jnp.concatenate([probs, t, pad], axis=1)


def _ksel_kernel(rt_ref, k_ref):
    t = rt_ref[:, 6:7]
    T = 2048
    iot = jax.lax.broadcasted_iota(jnp.int32, (1, T), 1).astype(_F32) + 0.5

    def order_stat(target):
        thr = 1.0 + 3.0 * iot / T
        counts = jnp.sum((t <= thr).astype(_F32), axis=0, keepdims=True)
        idx = jnp.sum((counts < target).astype(_F32))
        blo = 1.0 + 3.0 * (idx - 0.5) / T
        thr2 = blo + (3.0 / T) * iot / T
        counts2 = jnp.sum((t <= thr2).astype(_F32), axis=0, keepdims=True)
        idx2 = jnp.sum((counts2 < target).astype(_F32))
        return blo + (3.0 / T) * (idx2 + 0.5) / T

    med = 0.5 * (order_stat(B_TOK // 2) + order_stat(B_TOK // 2 + 1))
    k_ref[...] = jnp.zeros((1, 1), _F32) + jnp.clip(jnp.floor(med), 1.0, 4.0)


def _cf_kernel(xf_ref, inwT, inb, outwT, outb, fusewT, fuseb, out_ref):
    xf = xf_ref[...]
    xf16 = xf.astype(_BF16)
    qkv = _dot(xf16, inwT[...]) + inb[...]
    q = qkv[:, 0:FD].astype(_BF16)
    kk = qkv[:, FD:2 * FD].astype(_BF16)
    v = qkv[:, 2 * FD:3 * FD].astype(_BF16)

    G = 8
    R = G * NF
    ii = jax.lax.broadcasted_iota(jnp.int32, (R, R), 0) // NF
    jj = jax.lax.broadcasted_iota(jnp.int32, (R, R), 1) // NF
    mask = jnp.where(ii == jj, 0.0, -1e30).astype(_F32)
    scale = 1.0 / (FD ** 0.5)

    atts = []
    for g in range(xf.shape[0] // R):
        qg = q[g * R:(g + 1) * R, :]
        kg = kk[g * R:(g + 1) * R, :]
        vg = v[g * R:(g + 1) * R, :]
        s = jax.lax.dot_general(qg, kg, (((1,), (1,)), ((), ())),
                                preferred_element_type=_F32) * scale + mask
        sm = jnp.max(s, axis=1, keepdims=True)
        p = jnp.exp(s - sm)
        p = p / jnp.sum(p, axis=1, keepdims=True)
        atts.append(_dot(p.astype(_BF16), vg))

    att = jnp.concatenate(atts, axis=0)
    ao = _dot(att.astype(_BF16), outwT[...]) + outb[...]
    fused_in = (ao * xf).astype(_BF16)
    out_ref[...] = _dot(fused_in, fusewT[...]) + fuseb[...]


def _experts_kernel(x_ref, cf_ref, probs_ref, k_ref, sc_ref,
                    dew1, deb1, dew2, deb2, degw, degb,
                    hfw1, hfb1, hfw2, hfb2,
                    ltw1, ltb1, ltw2, ltb2,
                    spvw, spvb, spow, spob, spq1, spq1b, spq2, spq2b,
                    out_ref):
    x = x_ref[...]
    x16 = x.astype(_BF16)

    kv = k_ref[0]
    ps = [probs_ref[:, e:e + 1] for e in range(NE)]
    w = []
    mx = ps[0]
    for e in range(1, NE):
        mx = jnp.maximum(mx, ps[e])
    sels = []
    for e in range(NE):
        rank = jnp.zeros_like(ps[e])
        for e2 in range(NE):
            if e2 == e:
                continue
            gtr = (ps[e2] > ps[e]).astype(_F32)
            if e2 < e:
                gtr = gtr + ((ps[e2] == ps[e]).astype(_F32))
            rank = rank + gtr
        sels.append((rank < kv).astype(_F32))
    z = jnp.zeros_like(ps[0])
    ge = []
    for e in range(NE):
        g = sels[e] * jnp.exp(ps[e] - mx)
        ge.append(g)
        z = z + g
    for e in range(NE):
        w.append(ge[e] / z)

    h = jnp.maximum(_dotT(x16, dew1[...]) + deb1[...], 0.0)
    t = _dotT(h.astype(_BF16), dew2[...]) + deb2[...]
    g = jax.nn.sigmoid(_dotT(x16, degw[...]) + degb[...])
    acc = w[0] * cf_ref[...] + w[1] * (t + g * x)

    h = jnp.tanh(_dotT(x16, hfw1[...]) + hfb1[...])
    f = _dotT(h.astype(_BF16), hfw2[...]) + hfb2[...]
    acc = acc + w[2] * (x + (f - x) * x)

    zpre = _dotT(x16, ltw1[...]) + ltb1[...]
    h = jnp.where(zpre > 0.0, zpre, jnp.exp(zpre) - 1.0)
    t = _dotT(h.astype(_BF16), ltw2[...]) + ltb2[...]
    acc = acc + w[3] * (jnp.sign(x) * jnp.sqrt(jnp.abs(t * x) + 1e-8))

    v = _dotT(x16, spvw[...]) + spvb[...]
    xa = _dotT(v.astype(_BF16), spow[...]) + spob[...]
    h = jnp.maximum(_dotT((xa * x).astype(_BF16), spq1[...]) + spq1b[...], 0.0)
    sp = _dotT(h.astype(_BF16), spq2[...]) + spq2b[...]
    acc = acc + w[4] * sp

    bt = x.shape[0]
    zcol = jnp.zeros((bt, 1), dtype=_F32)
    xm = jnp.concatenate([zcol, x[:, :D - 1]], axis=1)
    xp = jnp.concatenate([x[:, 1:], zcol], axis=1)
    wa = jnp.zeros_like(x)
    for o in range(4):
        co = sc_ref[o * 3] * xm + sc_ref[o * 3 + 1] * x + sc_ref[o * 3 + 2] * xp
        ro = jnp.maximum(co + sc_ref[12 + o], 0.0)
        wa = wa + sc_ref[16 + o] * ro
    wgt = jax.nn.sigmoid(wa + sc_ref[20])
    acc = acc + w[5] * (x * wgt)

    out_ref[...] = acc


def _full(shape):
    return pl.BlockSpec(shape, lambda i: (0, 0))


@functools.partial(jax.jit, static_argnums=())
def kernel(x, es_patterns, es_w1, es_b1, es_w2, es_b2, rt_w1, rt_b1, rt_w2,
           rt_b2, kp_w1, kp_b1, kp_w2, kp_b2, cf_in_w, cf_in_b, cf_out_w,
           cf_out_b, cf_fuse_w, cf_fuse_b, de_w1, de_b1, de_w2, de_b2, de_gw,
           de_gb, hf_w1, hf_b1, hf_w2, hf_b2, lt_w1, lt_b1, lt_w2, lt_b2,
           sp_in_w, sp_in_b, sp_out_w, sp_out_b, sp_q1_w, sp_q1_b, sp_q2_w,
           sp_q2_b, tc_w, tc_b, tf_w, tf_b):
    f32 = _F32
    x = x.astype(f32)

    BT_R = 512
    grid_r = (B_TOK // BT_R,)
    rt_out = pl.pallas_call(
        _router_kernel,
        grid=grid_r,
        in_specs=[
            pl.BlockSpec((BT_R, D), lambda i: (i, 0)),
            _full((NE, D)),
            _full((D // 2, D)), _full((1, D // 2)),
            _full((NE, D // 2)), _full((1, NE)),
            _full((D // 2, D)),
            _full((D // 2, NE)), _full((1, D // 2)),
            _full((NE, D // 2)), _full((1, NE)),
            _full((16, NE)), _full((1, 16)), _full((16, 1)), _full((1, 1)),
        ],
        out_specs=pl.BlockSpec((BT_R, 8), lambda i: (i, 0)),
        out_shape=jax.ShapeDtypeStruct((B_TOK, 8), f32),
    )(x, es_patterns, es_w1,
      es_b1.reshape(1, -1), es_w2, es_b2.reshape(1, -1),
      rt_w1[:, :D], rt_w1[:, D:],
      rt_b1.reshape(1, -1), rt_w2, rt_b2.reshape(1, -1),
      kp_w1, kp_b1.reshape(1, -1),
      kp_w2.T, kp_b2.reshape(1, -1))

    kval = pl.pallas_call(
        _ksel_kernel,
        out_shape=jax.ShapeDtypeStruct((1, 1), f32),
    )(rt_out)

    xf = x.reshape(B_TOK * NF, FD)
    BT_C = 256 * NF
    cf_out = pl.pallas_call(
        _cf_kernel,
        grid=(B_TOK * NF // BT_C,),
        in_specs=[
            pl.BlockSpec((BT_C, FD), lambda i: (i, 0)),
            _full((FD, 3 * FD)), _full((1, 3 * FD)),
            _full((FD, FD)), _full((1, FD)),
            _full((FD, FD)), _full((1, FD)),
        ],
        out_specs=pl.BlockSpec((BT_C, FD), lambda i: (i, 0)),
        out_shape=jax.ShapeDtypeStruct((B_TOK * NF, FD), f32),
    )(xf, cf_in_w.T.astype(_BF16), cf_in_b.reshape(1, -1),
      cf_out_w.T.astype(_BF16), cf_out_b.reshape(1, -1),
      cf_fuse_w.T.astype(_BF16), cf_fuse_b.reshape(1, -1))
    cfr = cf_out.reshape(B_TOK, D)

    sc = jnp.concatenate([tc_w.reshape(-1), tc_b.reshape(-1),
                          tf_w.reshape(-1), tf_b.reshape(-1)]).astype(f32)
    BT_E = 256
    bspec = lambda shape: pl.BlockSpec(shape, lambda i: (0, 0))
    out = pl.pallas_call(
        _experts_kernel,
        grid=(B_TOK // BT_E,),
        in_specs=[
            pl.BlockSpec((BT_E, D), lambda i: (i, 0)),
            pl.BlockSpec((BT_E, D), lambda i: (i, 0)),
            pl.BlockSpec((BT_E, 8), lambda i: (i, 0)),
            pl.BlockSpec(memory_space=pltpu.SMEM),
            pl.BlockSpec(memory_space=pltpu.SMEM),
            bspec((H, D)), bspec((1, H)), bspec((D, H)), bspec((1, D)),
            bspec((D, D)), bspec((1, D)),
            bspec((H, D)), bspec((1, H)), bspec((D, H)), bspec((1, D)),
            bspec((H, D)), bspec((1, H)), bspec((D, H)), bspec((1, D)),
            bspec((D, D)), bspec((1, D)), bspec((D, D)), bspec((1, D)),
            bspec((D, D)), bspec((1, D)), bspec((D, D)), bspec((1, D)),
        ],
        out_specs=pl.BlockSpec((BT_E, D), lambda i: (i, 0)),
        out_shape=jax.ShapeDtypeStruct((B_TOK, D), f32),
    )(x, cfr, rt_out, kval.reshape(-1), sc,
      de_w1.astype(_BF16), de_b1.reshape(1, -1),
      de_w2.astype(_BF16), de_b2.reshape(1, -1),
      de_gw.astype(_BF16), de_gb.reshape(1, -1),
      hf_w1.astype(_BF16), hf_b1.reshape(1, -1),
      hf_w2.astype(_BF16), hf_b2.reshape(1, -1),
      lt_w1.astype(_BF16), lt_b1.reshape(1, -1),
      lt_w2.astype(_BF16), lt_b2.reshape(1, -1),
      sp_in_w[2 * D:, :].astype(_BF16), sp_in_b[2 * D:].reshape(1, -1),
      sp_out_w.astype(_BF16), sp_out_b.reshape(1, -1),
      sp_q1_w.astype(_BF16), sp_q1_b.reshape(1, -1),
      sp_q2_w.astype(_BF16), sp_q2_b.reshape(1, -1))
    return out
